# baseline (speedup 1.0000x reference)
import dataclasses
import functools

import jax
import jax.numpy as jnp
from jax import lax
from jax.experimental import pallas as pl
from jax.experimental.pallas import tpu as pltpu
from jax.experimental.pallas import tpu_sc as plsc

NUM_CORES = 2
NUM_SUBCORES = 16
NUM_WORKERS = NUM_CORES * NUM_SUBCORES
LANES = 16
L_PAD = 56
EOUT = 8
NBUF = 4
T_UNROLL = 10


@functools.partial(jax.jit, static_argnames=("B", "L", "Dw"))
def _pooled_lookup(idx, table_pk, B, L, Dw):
    b_per_w = B // NUM_WORKERS
    mesh = plsc.VectorSubcoreMesh(core_axis_name="c", subcore_axis_name="s")
    inv_l = jnp.float32(1.0 / L)
    n_chunks = b_per_w // EOUT

    cp = pltpu.CompilerParams()
    if "needs_layout_passes" in pltpu.CompilerParams.__dataclass_fields__:
        cp = dataclasses.replace(cp, needs_layout_passes=False)

    @functools.partial(
        pl.kernel,
        mesh=mesh,
        compiler_params=cp,
        out_type=jax.ShapeDtypeStruct((B, Dw), jnp.int32),
        scratch_types=[
            pltpu.VMEM((b_per_w, L_PAD), jnp.int32),
            pltpu.VMEM((NBUF, L_PAD, Dw), jnp.int32),
            pltpu.VMEM((EOUT, Dw), jnp.int32),
        ]
        + [pltpu.SemaphoreType.DMA] * NBUF,
    )
    def k(idx_hbm, table_hbm, out_hbm, idx_v, rows_v, obuf_v, *sems):
        wid = lax.axis_index("s") * NUM_CORES + lax.axis_index("c")
        pltpu.sync_copy(idx_hbm.at[wid], idx_v)

        def start_gather(e, slot):
            e = jnp.minimum(e, b_per_w - 1)
            pltpu.async_copy(
                table_hbm.at[idx_v.at[e]], rows_v.at[slot], sems[slot]
            )

        def wait_gather(slot):
            pltpu.make_async_copy(
                table_hbm.at[idx_v.at[0]], rows_v.at[slot], sems[slot]
            ).wait()

        def reduce_into(slot, i):
            buf = rows_v.at[slot]

            @pl.loop(0, Dw, step=LANES)
            def _dblock(db):
                def tblock(tb, carry):
                    acc_e, acc_o = carry
                    base = tb * T_UNROLL
                    for j in range(T_UNROLL):
                        w = buf[base + j, pl.ds(db, LANES)]
                        a, b = plsc.unpack(
                            plsc.bitcast(w, jnp.bfloat16),
                            format=plsc.PackFormat.INTERLEAVED,
                        )
                        acc_e = acc_e + a
                        acc_o = acc_o + b
                    return acc_e, acc_o

                zeros = jnp.zeros((LANES,), jnp.float32)
                acc_e, acc_o = lax.fori_loop(
                    0, L // T_UNROLL, tblock, (zeros, zeros)
                )
                pooled = plsc.pack(
                    acc_e * inv_l,
                    acc_o * inv_l,
                    format=plsc.PackFormat.INTERLEAVED,
                )
                obuf_v[i, pl.ds(db, LANES)] = plsc.bitcast(pooled, jnp.int32)

        for s in range(NBUF - 1):
            start_gather(s, s)

        @pl.loop(0, n_chunks)
        def _chunk(c):
            e0 = c * EOUT
            for q in range(EOUT):
                slot = q % NBUF
                wait_gather(slot)
                start_gather(e0 + q + NBUF - 1, (q + NBUF - 1) % NBUF)
                reduce_into(slot, q)

            pltpu.sync_copy(
                obuf_v, out_hbm.at[pl.ds(wid * b_per_w + e0, EOUT)]
            )

        for s in range(NBUF - 1):
            wait_gather(s)

    return k(idx, table_pk)


def kernel(input_ids, table):
    B, L = input_ids.shape
    V, D = table.shape
    idx = input_ids.astype(jnp.int32)
    if L_PAD != L:
        idx = jnp.pad(idx, ((0, 0), (0, L_PAD - L)))
    idx = idx.reshape(NUM_WORKERS, B // NUM_WORKERS, L_PAD)
    table_pk = lax.bitcast_convert_type(
        table.astype(jnp.bfloat16).reshape(V, D // 2, 2), jnp.int32
    )
    out_pk = _pooled_lookup(idx, table_pk, B, L, D // 2)
    out_bf = lax.bitcast_convert_type(out_pk, jnp.bfloat16).reshape(B, D)
    return out_bf.astype(jnp.float32)
